# baseline (speedup 1.0000x reference)
import functools

import jax
import jax.numpy as jnp
from jax import lax
from jax.experimental import pallas as pl
from jax.experimental.pallas import tpu as pltpu
from jax.experimental.pallas import tpu_sc as plsc


def _make_gather(N, D, NW, NC, n_chunks, C):
    b_per_w = N // NW
    mesh = plsc.VectorSubcoreMesh(core_axis_name="c", subcore_axis_name="s")

    @functools.partial(
        pl.kernel,
        mesh=mesh,
        compiler_params=pltpu.CompilerParams(use_tc_tiling_on_sc=False),
        out_type=jax.ShapeDtypeStruct((N, D), jnp.float32),
        scratch_types=[
            pltpu.VMEM((n_chunks, C), jnp.int32),
            pltpu.VMEM((C, D), jnp.float32),
            pltpu.SemaphoreType.DMA,
        ],
    )
    def k(idx_hbm, table_hbm, out_hbm, idx_v, rows_v, sem):
        wid = lax.axis_index("s") * NC + lax.axis_index("c")
        base = wid * b_per_w
        pltpu.sync_copy(idx_hbm.at[wid], idx_v)

        def body(j, carry):
            pltpu.async_copy(table_hbm.at[idx_v.at[j]], rows_v, sem).wait()
            pltpu.sync_copy(rows_v, out_hbm.at[pl.ds(base + j * C, C)])
            return carry

        lax.fori_loop(0, n_chunks, body, 0)

    return k


def kernel(input_batch, table):
    B, H = input_batch.shape
    V, D = table.shape
    flat = input_batch.reshape(-1).astype(jnp.int32)
    N = flat.shape[0]

    info = plsc.get_sparse_core_info()
    NC, NS = info.num_cores, info.num_subcores
    NW = NC * NS
    C = 128
    b_per_w = N // NW
    n_chunks = b_per_w // C

    idx3 = flat.reshape(NW, n_chunks, C)
    out = _make_gather(N, D, NW, NC, n_chunks, C)(idx3, table)
    return out.reshape(B, H, D)


# speedup vs baseline: 1.0365x; 1.0365x over previous
import functools

import jax
import jax.numpy as jnp
from jax import lax
from jax.experimental import pallas as pl
from jax.experimental.pallas import tpu as pltpu
from jax.experimental.pallas import tpu_sc as plsc


def _make_gather(N, D, NW, NC, G):
    b_per_w = N // NW
    n_groups = b_per_w // G
    mesh = plsc.VectorSubcoreMesh(core_axis_name="c", subcore_axis_name="s")

    @functools.partial(
        pl.kernel,
        mesh=mesh,
        compiler_params=pltpu.CompilerParams(use_tc_tiling_on_sc=False),
        out_type=jax.ShapeDtypeStruct((N, D), jnp.float32),
        scratch_types=[
            pltpu.VMEM((b_per_w,), jnp.int32),
            pltpu.VMEM((G, D), jnp.float32),
            pltpu.SemaphoreType.DMA,
        ],
    )
    def k(idx_hbm, table_hbm, out_hbm, idx_v, rows_v, sem):
        wid = lax.axis_index("s") * NC + lax.axis_index("c")
        base = wid * b_per_w
        pltpu.sync_copy(idx_hbm.at[wid], idx_v)

        def body(g, carry):
            pltpu.async_copy(
                table_hbm.at[idx_v.at[pl.ds(g * G, G)]], rows_v, sem
            ).wait()
            pltpu.sync_copy(rows_v, out_hbm.at[pl.ds(base + g * G, G)])
            return carry

        lax.fori_loop(0, n_groups, body, 0)

    return k


def kernel(input_batch, table):
    B, H = input_batch.shape
    V, D = table.shape
    flat = input_batch.reshape(-1).astype(jnp.int32)
    N = flat.shape[0]

    info = plsc.get_sparse_core_info()
    NC, NS = info.num_cores, info.num_subcores
    NW = NC * NS
    G = 640
    b_per_w = N // NW

    idx2 = flat.reshape(NW, b_per_w)
    out = _make_gather(N, D, NW, NC, G)(idx2, table)
    return out.reshape(B, H, D)


# speedup vs baseline: 1.0406x; 1.0040x over previous
import functools

import jax
import jax.numpy as jnp
from jax import lax
from jax.experimental import pallas as pl
from jax.experimental.pallas import tpu as pltpu
from jax.experimental.pallas import tpu_sc as plsc


def _make_gather(N, D, NW, NC, G):
    b_per_w = N // NW
    n_groups = b_per_w // G
    T = n_groups // 2
    mesh = plsc.VectorSubcoreMesh(core_axis_name="c", subcore_axis_name="s")

    @functools.partial(
        pl.kernel,
        mesh=mesh,
        compiler_params=pltpu.CompilerParams(use_tc_tiling_on_sc=False),
        out_type=jax.ShapeDtypeStruct((N, D), jnp.float32),
        scratch_types=[
            pltpu.VMEM((b_per_w,), jnp.int32),
            pltpu.VMEM((G, D), jnp.float32),
            pltpu.VMEM((G, D), jnp.float32),
            pltpu.SemaphoreType.DMA,
            pltpu.SemaphoreType.DMA,
            pltpu.SemaphoreType.DMA,
            pltpu.SemaphoreType.DMA,
        ],
    )
    def k(idx_hbm, table_hbm, out_hbm, idx_v, rows0, rows1, sg0, sg1, sw0, sw1):
        wid = lax.axis_index("s") * NC + lax.axis_index("c")
        base = wid * b_per_w
        pltpu.sync_copy(idx_hbm.at[wid], idx_v)

        def gather(g, rows, sem):
            pltpu.async_copy(table_hbm.at[idx_v.at[pl.ds(g * G, G)]], rows, sem)

        def gather_wait(rows, sem):
            pltpu.make_async_copy(
                table_hbm.at[idx_v.at[pl.ds(0, G)]], rows, sem
            ).wait()

        def write(g, rows, sem):
            pltpu.async_copy(rows, out_hbm.at[pl.ds(base + g * G, G)], sem)

        def write_wait(rows, sem):
            pltpu.make_async_copy(rows, out_hbm.at[pl.ds(base, G)], sem).wait()

        gather(0, rows0, sg0)

        def body(t, carry):
            @pl.when(t > 0)
            def _():
                write_wait(rows1, sw1)

            gather(2 * t + 1, rows1, sg1)
            gather_wait(rows0, sg0)
            write(2 * t, rows0, sw0)
            gather_wait(rows1, sg1)

            @pl.when(t < T - 1)
            def _():
                write_wait(rows0, sw0)
                gather(2 * t + 2, rows0, sg0)

            write(2 * t + 1, rows1, sw1)
            return carry

        lax.fori_loop(0, T, body, 0)
        write_wait(rows0, sw0)
        write_wait(rows1, sw1)

    return k


def kernel(input_batch, table):
    B, H = input_batch.shape
    V, D = table.shape
    flat = input_batch.reshape(-1).astype(jnp.int32)
    N = flat.shape[0]

    info = plsc.get_sparse_core_info()
    NC, NS = info.num_cores, info.num_subcores
    NW = NC * NS
    G = 640
    b_per_w = N // NW

    idx2 = flat.reshape(NW, b_per_w)
    out = _make_gather(N, D, NW, NC, G)(idx2, table)
    return out.reshape(B, H, D)


# speedup vs baseline: 1.0439x; 1.0032x over previous
import functools

import jax
import jax.numpy as jnp
from jax import lax
from jax.experimental import pallas as pl
from jax.experimental.pallas import tpu as pltpu
from jax.experimental.pallas import tpu_sc as plsc


def _make_gather(N, D, NW, NC, C, NB):
    b_per_w = N // NW
    n_chunks = b_per_w // C
    T = n_chunks // NB
    mesh = plsc.VectorSubcoreMesh(core_axis_name="c", subcore_axis_name="s")

    scratch = [pltpu.VMEM((b_per_w,), jnp.int32)]
    scratch += [pltpu.VMEM((C, D), jnp.float32) for _ in range(NB)]
    scratch += [pltpu.SemaphoreType.DMA for _ in range(NB)]
    scratch += [pltpu.SemaphoreType.DMA]

    @functools.partial(
        pl.kernel,
        mesh=mesh,
        compiler_params=pltpu.CompilerParams(use_tc_tiling_on_sc=False),
        out_type=jax.ShapeDtypeStruct((N, D), jnp.float32),
        scratch_types=scratch,
    )
    def k(idx_hbm, table_hbm, out_hbm, idx_v, *rest):
        bufs = rest[:NB]
        sg = rest[NB : 2 * NB]
        sw = rest[2 * NB]
        wid = lax.axis_index("s") * NC + lax.axis_index("c")
        base = wid * b_per_w
        pltpu.sync_copy(idx_hbm.at[wid], idx_v)

        def gather(j, b):
            pltpu.async_copy(
                table_hbm.at[idx_v.at[pl.ds(j * C, C)]], bufs[b], sg[b]
            )

        def gather_wait(b):
            pltpu.make_async_copy(
                table_hbm.at[idx_v.at[pl.ds(0, C)]], bufs[b], sg[b]
            ).wait()

        def write(j, b):
            pltpu.async_copy(bufs[b], out_hbm.at[pl.ds(base + j * C, C)], sw)

        def write_wait(b):
            pltpu.make_async_copy(bufs[b], out_hbm.at[pl.ds(base, C)], sw).wait()

        for b in range(NB):
            gather(b, b)

        def body(t, carry):
            for b in range(NB):
                j = t * NB + b
                gather_wait(b)
                write(j, b)
                write_wait(b)

                @pl.when(t < T - 1)
                def _():
                    gather(j + NB, b)

            return carry

        lax.fori_loop(0, T, body, 0)

    return k


def kernel(input_batch, table):
    B, H = input_batch.shape
    V, D = table.shape
    flat = input_batch.reshape(-1).astype(jnp.int32)
    N = flat.shape[0]

    info = plsc.get_sparse_core_info()
    NC, NS = info.num_cores, info.num_subcores
    NW = NC * NS
    C = 160
    NB = 8
    b_per_w = N // NW

    idx2 = flat.reshape(NW, b_per_w)
    out = _make_gather(N, D, NW, NC, C, NB)(idx2, table)
    return out.reshape(B, H, D)
